# baseline (speedup 1.0000x reference)
import functools

import jax
import jax.numpy as jnp
from jax import lax
from jax.experimental import pallas as pl
from jax.experimental.pallas import tpu as pltpu
from jax.experimental.pallas import tpu_sc as plsc

_B = 1024
_WALK = 50
_NEG = 20
_D = 16
_NTOT = _B * (1 + _WALK + _NEG)
_NW = 32
_PER_W = 2304
_NPAD = _NW * _PER_W
_PER_E = _PER_W * _D
_CHUNK_E = 4608
_NCHUNK_E = _PER_E // _CHUNK_E


def _sc_gather(flat_t, eidx):
    mesh = plsc.VectorSubcoreMesh(core_axis_name="c", subcore_axis_name="s")

    @functools.partial(
        pl.kernel,
        mesh=mesh,
        out_type=jax.ShapeDtypeStruct((_NPAD * _D,), jnp.float32),
        scratch_types=[
            pltpu.VMEM((_PER_E,), jnp.int32),
            pltpu.VMEM((_PER_E,), jnp.float32),
            pltpu.SemaphoreType.DMA,
        ],
        compiler_params=pltpu.CompilerParams(use_tc_tiling_on_sc=False),
    )
    def k(tab_hbm, eidx_hbm, out_hbm, idx_v, vals_v, sem):
        nc = 2
        wid = lax.axis_index("s") * nc + lax.axis_index("c")
        base = wid * _PER_E
        pltpu.sync_copy(eidx_hbm.at[pl.ds(base, _PER_E)], idx_v)
        copies = []
        for c in range(_NCHUNK_E):
            copies.append(
                pltpu.async_copy(
                    tab_hbm.at[idx_v.at[pl.ds(c * _CHUNK_E, _CHUNK_E)]],
                    vals_v.at[pl.ds(c * _CHUNK_E, _CHUNK_E)],
                    sem,
                )
            )
        for cp in copies:
            cp.wait()
        pltpu.sync_copy(vals_v, out_hbm.at[pl.ds(base, _PER_E)])

    return k(flat_t, eidx)


def _dot(a, b):
    return jnp.dot(a, b, precision=lax.Precision.HIGHEST,
                   preferred_element_type=jnp.float32)


def _clip_scale(ss):
    return jnp.minimum(1.0, lax.rsqrt(jnp.maximum(ss, 1e-24)))


def _stats_body(nodes_ref, walks_ref, negs_ref, s50_ref, s50t_ref, u50_ref,
                s20_ref, s20t_ref, u20_ref,
                net_ref, loss_ref):
    e = nodes_ref[...]
    ss = jnp.sum(e * e, axis=1, keepdims=True)
    ne = e * _clip_scale(ss)
    net_ref[...] = jnp.transpose(ne)

    w = walks_ref[...]
    ssw = _dot(w * w, s50_ref[...])
    cw = w * _dot(_clip_scale(ssw), s50t_ref[...])
    net50 = _dot(ne, u50_ref[...])
    wsum = jnp.sum(cw * net50, axis=1, keepdims=True)

    g = negs_ref[...]
    ssn = _dot(g * g, s20_ref[...])
    cg = g * _dot(_clip_scale(ssn), s20t_ref[...])
    net20 = _dot(ne, u20_ref[...])
    sim = _dot(cg * net20, s20_ref[...])
    nsum = jnp.sum(jnp.exp(sim), axis=1, keepdims=True)
    loss_ref[0, 0] = jnp.sum(jnp.log(nsum) - wsum)


def _edge_body(nei_ref, net_ref, out_ref):
    out_ref[...] = nei_ref[...] * net_ref[...][None, :, :]


def _seg_onehot(width, d):
    r = lax.broadcasted_iota(jnp.int32, (width * d, width), 0) // d
    c = lax.broadcasted_iota(jnp.int32, (width * d, width), 1)
    return (r == c).astype(jnp.float32)


def _tile_onehot(n_lanes, d):
    r = lax.broadcasted_iota(jnp.int32, (d, n_lanes), 0)
    c = lax.broadcasted_iota(jnp.int32, (d, n_lanes), 1) % d
    return (r == c).astype(jnp.float32)


def kernel(nodes, walks, neg_samples, node_embedding_var):
    idx_all = jnp.concatenate(
        [nodes, walks.reshape(-1), neg_samples.reshape(-1),
         jnp.zeros((_NPAD - _NTOT,), jnp.int32)]
    )
    eidx = (idx_all[:, None]
            + (jnp.arange(_D, dtype=jnp.int32) * 1000000)[None, :]).reshape(-1)
    flat_t = jnp.transpose(node_embedding_var).reshape(-1)
    gathered = _sc_gather(flat_t, eidx)
    nodes_g = gathered[:_B * _D].reshape(_B, _D)
    walks_v = gathered[_B * _D:_B * (1 + _WALK) * _D].reshape(_B, _WALK * _D)
    negs_v = gathered[_B * (1 + _WALK) * _D:_NTOT * _D].reshape(_B, _NEG * _D)

    s50 = _seg_onehot(_WALK, _D)
    s50t = s50.T
    u50 = _tile_onehot(_WALK * _D, _D)
    s20 = _seg_onehot(_NEG, _D)
    s20t = s20.T
    u20 = _tile_onehot(_NEG * _D, _D)

    net, loss = pl.pallas_call(
        _stats_body,
        out_shape=(
            jax.ShapeDtypeStruct((_D, _B), jnp.float32),
            jax.ShapeDtypeStruct((1, 1), jnp.float32),
        ),
        out_specs=(
            pl.BlockSpec(memory_space=pltpu.VMEM),
            pl.BlockSpec(memory_space=pltpu.SMEM),
        ),
    )(nodes_g, walks_v, negs_v, s50, s50t, u50, s20, s20t, u20)

    ne = jnp.transpose(net)
    nei = ne.reshape(_B, _D, 1)
    edge_t = pl.pallas_call(
        _edge_body,
        grid=(16,),
        in_specs=[
            pl.BlockSpec((64, _D, 1), lambda i: (i, 0, 0)),
            pl.BlockSpec((_D, _B), lambda i: (0, 0)),
        ],
        out_specs=pl.BlockSpec((64, _D, _B), lambda i: (i, 0, 0)),
        out_shape=jax.ShapeDtypeStruct((_B, _D, _B), jnp.float32),
    )(nei, net)
    edge = jnp.transpose(edge_t, (0, 2, 1))
    return loss[0, 0], ne, edge


# speedup vs baseline: 6.3580x; 6.3580x over previous
import functools

import jax
import jax.numpy as jnp
from jax import lax
from jax.experimental import pallas as pl
from jax.experimental.pallas import tpu as pltpu
from jax.experimental.pallas import tpu_sc as plsc

_B = 1024
_WALK = 50
_NEG = 20
_D = 16
_NTOT = _B * (1 + _WALK + _NEG)
_NW = 32
_PER_W = 2304
_NPAD = _NW * _PER_W
_PER_E = _PER_W * _D
_CHUNK_E = 4608
_NCHUNK_E = _PER_E // _CHUNK_E


def _sc_gather(flat_t, eidx):
    mesh = plsc.VectorSubcoreMesh(core_axis_name="c", subcore_axis_name="s")

    @functools.partial(
        pl.kernel,
        mesh=mesh,
        out_type=jax.ShapeDtypeStruct((_NPAD * _D,), jnp.float32),
        scratch_types=[
            pltpu.VMEM((_PER_E,), jnp.int32),
            pltpu.VMEM((_PER_E,), jnp.float32),
            pltpu.SemaphoreType.DMA,
        ],
        compiler_params=pltpu.CompilerParams(use_tc_tiling_on_sc=False),
    )
    def k(tab_hbm, eidx_hbm, out_hbm, idx_v, vals_v, sem):
        nc = 2
        wid = lax.axis_index("s") * nc + lax.axis_index("c")
        base = wid * _PER_E
        pltpu.sync_copy(eidx_hbm.at[pl.ds(base, _PER_E)], idx_v)
        copies = []
        for c in range(_NCHUNK_E):
            copies.append(
                pltpu.async_copy(
                    tab_hbm.at[idx_v.at[pl.ds(c * _CHUNK_E, _CHUNK_E)]],
                    vals_v.at[pl.ds(c * _CHUNK_E, _CHUNK_E)],
                    sem,
                )
            )
        for cp in copies:
            cp.wait()
        pltpu.sync_copy(vals_v, out_hbm.at[pl.ds(base, _PER_E)])

    return k(flat_t, eidx)


_ROW_S = 1 << 20
_DT_C = 65536
_DT_NC = (1000000 + _DT_C - 1) // _DT_C


def _detile_body(src_ref, dst_ref, sem):
    c = pl.program_id(0)
    copies = []
    for d in range(_D):
        copies.append(
            pltpu.async_copy(
                src_ref.at[d],
                dst_ref.at[pl.ds(d * _ROW_S + c * _DT_C, _DT_C)],
                sem,
            )
        )
    for cp in copies:
        cp.wait()


def _detile(table):
    tab_t = jnp.transpose(table)
    return pl.pallas_call(
        _detile_body,
        grid=(_DT_NC,),
        in_specs=[pl.BlockSpec((_D, _DT_C), lambda c: (0, c))],
        out_specs=pl.BlockSpec(memory_space=pltpu.MemorySpace.HBM),
        out_shape=jax.ShapeDtypeStruct((_D * _ROW_S,), jnp.float32),
        scratch_shapes=[pltpu.SemaphoreType.DMA],
    )(tab_t)


def _dot(a, b):
    return jnp.dot(a, b, precision=lax.Precision.HIGHEST,
                   preferred_element_type=jnp.float32)


def _clip_scale(ss):
    return jnp.minimum(1.0, lax.rsqrt(jnp.maximum(ss, 1e-24)))


def _stats_body(nodes_ref, walks_ref, negs_ref, s50_ref, s50t_ref, u50_ref,
                s20_ref, s20t_ref, u20_ref,
                net_ref, loss_ref):
    e = nodes_ref[...]
    ss = jnp.sum(e * e, axis=1, keepdims=True)
    ne = e * _clip_scale(ss)
    net_ref[...] = jnp.transpose(ne)

    w = walks_ref[...]
    ssw = _dot(w * w, s50_ref[...])
    cw = w * _dot(_clip_scale(ssw), s50t_ref[...])
    net50 = _dot(ne, u50_ref[...])
    wsum = jnp.sum(cw * net50, axis=1, keepdims=True)

    g = negs_ref[...]
    ssn = _dot(g * g, s20_ref[...])
    cg = g * _dot(_clip_scale(ssn), s20t_ref[...])
    net20 = _dot(ne, u20_ref[...])
    sim = _dot(cg * net20, s20_ref[...])
    nsum = jnp.sum(jnp.exp(sim), axis=1, keepdims=True)
    loss_ref[0, 0] = jnp.sum(jnp.log(nsum) - wsum)


def _edge_body(nei_ref, net_ref, out_ref):
    out_ref[...] = nei_ref[...] * net_ref[...][None, :, :]


def _seg_onehot(width, d):
    r = lax.broadcasted_iota(jnp.int32, (width * d, width), 0) // d
    c = lax.broadcasted_iota(jnp.int32, (width * d, width), 1)
    return (r == c).astype(jnp.float32)


def _tile_onehot(n_lanes, d):
    r = lax.broadcasted_iota(jnp.int32, (d, n_lanes), 0)
    c = lax.broadcasted_iota(jnp.int32, (d, n_lanes), 1) % d
    return (r == c).astype(jnp.float32)


def kernel(nodes, walks, neg_samples, node_embedding_var):
    idx_all = jnp.concatenate(
        [nodes, walks.reshape(-1), neg_samples.reshape(-1),
         jnp.zeros((_NPAD - _NTOT,), jnp.int32)]
    )
    eidx = (idx_all[:, None]
            + (jnp.arange(_D, dtype=jnp.int32) * _ROW_S)[None, :]).reshape(-1)
    flat_t = _detile(node_embedding_var)
    gathered = _sc_gather(flat_t, eidx)
    nodes_g = gathered[:_B * _D].reshape(_B, _D)
    walks_v = gathered[_B * _D:_B * (1 + _WALK) * _D].reshape(_B, _WALK * _D)
    negs_v = gathered[_B * (1 + _WALK) * _D:_NTOT * _D].reshape(_B, _NEG * _D)

    s50 = _seg_onehot(_WALK, _D)
    s50t = s50.T
    u50 = _tile_onehot(_WALK * _D, _D)
    s20 = _seg_onehot(_NEG, _D)
    s20t = s20.T
    u20 = _tile_onehot(_NEG * _D, _D)

    net, loss = pl.pallas_call(
        _stats_body,
        out_shape=(
            jax.ShapeDtypeStruct((_D, _B), jnp.float32),
            jax.ShapeDtypeStruct((1, 1), jnp.float32),
        ),
        out_specs=(
            pl.BlockSpec(memory_space=pltpu.VMEM),
            pl.BlockSpec(memory_space=pltpu.SMEM),
        ),
    )(nodes_g, walks_v, negs_v, s50, s50t, u50, s20, s20t, u20)

    ne = jnp.transpose(net)
    nei = ne.reshape(_B, _D, 1)
    edge_t = pl.pallas_call(
        _edge_body,
        grid=(16,),
        in_specs=[
            pl.BlockSpec((64, _D, 1), lambda i: (i, 0, 0)),
            pl.BlockSpec((_D, _B), lambda i: (0, 0)),
        ],
        out_specs=pl.BlockSpec((64, _D, _B), lambda i: (i, 0, 0)),
        out_shape=jax.ShapeDtypeStruct((_B, _D, _B), jnp.float32),
    )(nei, net)
    edge = jnp.transpose(edge_t, (0, 2, 1))
    return loss[0, 0], ne, edge


# speedup vs baseline: 6.5822x; 1.0353x over previous
import functools

import jax
import jax.numpy as jnp
from jax import lax
from jax.experimental import pallas as pl
from jax.experimental.pallas import tpu as pltpu
from jax.experimental.pallas import tpu_sc as plsc

_B = 1024
_WALK = 50
_NEG = 20
_D = 16
_NTOT = _B * (1 + _WALK + _NEG)
_NW = 32
_PER_W = 2304
_NPAD = _NW * _PER_W
_ROW_S = 1 << 20


def _sc_gather(flat_t, idx):
    mesh = plsc.VectorSubcoreMesh(core_axis_name="c", subcore_axis_name="s")

    @functools.partial(
        pl.kernel,
        mesh=mesh,
        out_type=jax.ShapeDtypeStruct((_D * _NPAD,), jnp.float32),
        scratch_types=[
            pltpu.VMEM((_PER_W,), jnp.int32),
            pltpu.VMEM((_D * _PER_W,), jnp.float32),
            pltpu.SemaphoreType.DMA,
            pltpu.SemaphoreType.DMA,
        ],
        compiler_params=pltpu.CompilerParams(use_tc_tiling_on_sc=False),
    )
    def k(tab_hbm, idx_hbm, out_hbm, idx_v, vals_v, gsem, wsem):
        nc = 2
        wid = lax.axis_index("s") * nc + lax.axis_index("c")
        base = wid * _PER_W
        pltpu.sync_copy(idx_hbm.at[pl.ds(base, _PER_W)], idx_v)
        gathers = []
        for d in range(_D):
            gathers.append(
                pltpu.async_copy(
                    tab_hbm.at[pl.ds(d * _ROW_S, _ROW_S)].at[idx_v],
                    vals_v.at[pl.ds(d * _PER_W, _PER_W)],
                    gsem,
                )
            )
        for g in gathers:
            g.wait()
        writes = []
        for d in range(_D):
            writes.append(
                pltpu.async_copy(
                    vals_v.at[pl.ds(d * _PER_W, _PER_W)],
                    out_hbm.at[pl.ds(d * _NPAD + base, _PER_W)],
                    wsem,
                )
            )
        for w in writes:
            w.wait()

    return k(flat_t, idx)


_DT_C = 65536
_DT_NC = (1000000 + _DT_C - 1) // _DT_C


def _detile_body(src_ref, dst_ref, sem):
    c = pl.program_id(0)
    copies = []
    for d in range(_D):
        copies.append(
            pltpu.async_copy(
                src_ref.at[d],
                dst_ref.at[pl.ds(d * _ROW_S + c * _DT_C, _DT_C)],
                sem,
            )
        )
    for cp in copies:
        cp.wait()


def _detile(table):
    tab_t = jnp.transpose(table)
    return pl.pallas_call(
        _detile_body,
        grid=(_DT_NC,),
        in_specs=[pl.BlockSpec((_D, _DT_C), lambda c: (0, c))],
        out_specs=pl.BlockSpec(memory_space=pltpu.MemorySpace.HBM),
        out_shape=jax.ShapeDtypeStruct((_D * _ROW_S,), jnp.float32),
        scratch_shapes=[pltpu.SemaphoreType.DMA],
    )(tab_t)


def _clip_scale(ss):
    return jnp.minimum(1.0, lax.rsqrt(jnp.maximum(ss, 1e-24)))


def _stats_body(nt_ref, wt_ref, gt_ref, net_ref, loss_ref):
    nt = nt_ref[...]
    ssn = jnp.sum(nt * nt, axis=0, keepdims=True)
    net = nt * _clip_scale(ssn)
    net_ref[...] = net

    wt = wt_ref[...]
    ssw = jnp.sum(wt * wt, axis=0, keepdims=True)
    cw = wt * _clip_scale(ssw)
    cwsum = cw[:, :_B]
    for w in range(1, _WALK):
        cwsum = cwsum + cw[:, w * _B:(w + 1) * _B]
    wsum_total = jnp.sum(net * cwsum)

    gt = gt_ref[...]
    ssg = jnp.sum(gt * gt, axis=0, keepdims=True)
    cg = gt * _clip_scale(ssg)
    nsum = jnp.zeros((1, _B), jnp.float32)
    for n in range(_NEG):
        s_n = jnp.sum(cg[:, n * _B:(n + 1) * _B] * net, axis=0, keepdims=True)
        nsum = nsum + jnp.exp(s_n)
    loss_ref[0, 0] = jnp.sum(jnp.log(nsum)) - wsum_total


def _edge_body(nei_ref, net_ref, out_ref):
    out_ref[...] = nei_ref[...][:, :, None] * net_ref[...][None, :, :]


def kernel(nodes, walks, neg_samples, node_embedding_var):
    idx_all = jnp.concatenate(
        [nodes, jnp.transpose(walks).reshape(-1),
         jnp.transpose(neg_samples).reshape(-1),
         jnp.zeros((_NPAD - _NTOT,), jnp.int32)]
    )
    flat_t = _detile(node_embedding_var)
    gathered = _sc_gather(flat_t, idx_all)
    g2 = gathered.reshape(_D, _NPAD)
    nt = g2[:, :_B]
    wt = g2[:, _B:_B * (1 + _WALK)]
    gt = g2[:, _B * (1 + _WALK):_NTOT]

    net, loss = pl.pallas_call(
        _stats_body,
        out_shape=(
            jax.ShapeDtypeStruct((_D, _B), jnp.float32),
            jax.ShapeDtypeStruct((1, 1), jnp.float32),
        ),
        out_specs=(
            pl.BlockSpec(memory_space=pltpu.VMEM),
            pl.BlockSpec(memory_space=pltpu.SMEM),
        ),
    )(nt, wt, gt)

    ne = jnp.transpose(net)
    edge_t = pl.pallas_call(
        _edge_body,
        grid=(16,),
        in_specs=[
            pl.BlockSpec((64, _D), lambda i: (i, 0)),
            pl.BlockSpec((_D, _B), lambda i: (0, 0)),
        ],
        out_specs=pl.BlockSpec((64, _D, _B), lambda i: (i, 0, 0)),
        out_shape=jax.ShapeDtypeStruct((_B, _D, _B), jnp.float32),
    )(ne, net)
    edge = jnp.transpose(edge_t, (0, 2, 1))
    return loss[0, 0], ne, edge


# speedup vs baseline: 9.1023x; 1.3829x over previous
import functools

import jax
import jax.numpy as jnp
from jax import lax
from jax.experimental import pallas as pl
from jax.experimental.pallas import tpu as pltpu
from jax.experimental.pallas import tpu_sc as plsc

_B = 1024
_WALK = 50
_NEG = 20
_D = 16
_NTOT = _B * (1 + _WALK + _NEG)
_NW = 32
_PER_W = 2304
_NPAD = _NW * _PER_W
_ROW_S = 1 << 20


_PER_E = _D * _NPAD // _NW
_CHUNK_E = 4608
_NCHUNK_E = _PER_E // _CHUNK_E


def _sc_gather(flat_t, eidx):
    mesh = plsc.VectorSubcoreMesh(core_axis_name="c", subcore_axis_name="s")

    @functools.partial(
        pl.kernel,
        mesh=mesh,
        out_type=jax.ShapeDtypeStruct((_D * _NPAD,), jnp.float32),
        scratch_types=[
            pltpu.VMEM((_PER_E,), jnp.int32),
            pltpu.VMEM((_PER_E,), jnp.float32),
            pltpu.SemaphoreType.DMA,
        ],
        compiler_params=pltpu.CompilerParams(use_tc_tiling_on_sc=False),
    )
    def k(tab_hbm, eidx_hbm, out_hbm, idx_v, vals_v, sem):
        nc = 2
        wid = lax.axis_index("s") * nc + lax.axis_index("c")
        base = wid * _PER_E
        pltpu.sync_copy(eidx_hbm.at[pl.ds(base, _PER_E)], idx_v)
        copies = []
        for c in range(_NCHUNK_E):
            copies.append(
                pltpu.async_copy(
                    tab_hbm.at[idx_v.at[pl.ds(c * _CHUNK_E, _CHUNK_E)]],
                    vals_v.at[pl.ds(c * _CHUNK_E, _CHUNK_E)],
                    sem,
                )
            )
        for cp in copies:
            cp.wait()
        pltpu.sync_copy(vals_v, out_hbm.at[pl.ds(base, _PER_E)])

    return k(flat_t, eidx)


_DT_C = 65536
_DT_NC = (1000000 + _DT_C - 1) // _DT_C


_EIDX_R = _NPAD // 128


def _detile_body(src_ref, idx_ref, dst_ref, eidx_ref, sem):
    c = pl.program_id(0)
    copies = []
    for d in range(_D):
        copies.append(
            pltpu.async_copy(
                src_ref.at[d],
                dst_ref.at[pl.ds(d * _ROW_S + c * _DT_C, _DT_C)],
                sem,
            )
        )
    eidx_ref[...] = idx_ref[...] + c * _ROW_S
    for cp in copies:
        cp.wait()


def _detile(table, idx2):
    tab_t = jnp.transpose(table)
    return pl.pallas_call(
        _detile_body,
        grid=(_DT_NC,),
        in_specs=[
            pl.BlockSpec((_D, _DT_C), lambda c: (0, c)),
            pl.BlockSpec((_EIDX_R, 128), lambda c: (0, 0)),
        ],
        out_specs=(
            pl.BlockSpec(memory_space=pltpu.MemorySpace.HBM),
            pl.BlockSpec((_EIDX_R, 128), lambda c: (c, 0)),
        ),
        out_shape=(
            jax.ShapeDtypeStruct((_D * _ROW_S,), jnp.float32),
            jax.ShapeDtypeStruct((_D * _EIDX_R, 128), jnp.int32),
        ),
        scratch_shapes=[pltpu.SemaphoreType.DMA],
    )(tab_t, idx2)


def _clip_scale(ss):
    return jnp.minimum(1.0, lax.rsqrt(jnp.maximum(ss, 1e-24)))


def _stats_body(nt_ref, wt_ref, gt_ref, net_ref, loss_ref):
    nt = nt_ref[...]
    ssn = jnp.sum(nt * nt, axis=0, keepdims=True)
    net = nt * _clip_scale(ssn)
    net_ref[...] = net

    wt = wt_ref[...]
    ssw = jnp.sum(wt * wt, axis=0, keepdims=True)
    cw = wt * _clip_scale(ssw)
    cwsum = cw[:, :_B]
    for w in range(1, _WALK):
        cwsum = cwsum + cw[:, w * _B:(w + 1) * _B]
    wsum_total = jnp.sum(net * cwsum)

    gt = gt_ref[...]
    ssg = jnp.sum(gt * gt, axis=0, keepdims=True)
    cg = gt * _clip_scale(ssg)
    nsum = jnp.zeros((1, _B), jnp.float32)
    for n in range(_NEG):
        s_n = jnp.sum(cg[:, n * _B:(n + 1) * _B] * net, axis=0, keepdims=True)
        nsum = nsum + jnp.exp(s_n)
    loss_ref[0, 0] = jnp.sum(jnp.log(nsum)) - wsum_total


def _edge_body(nei_ref, net_ref, out_ref):
    out_ref[...] = nei_ref[...][:, :, None] * net_ref[...][None, :, :]


def kernel(nodes, walks, neg_samples, node_embedding_var):
    idx_all = jnp.concatenate(
        [nodes, jnp.transpose(walks).reshape(-1),
         jnp.transpose(neg_samples).reshape(-1),
         jnp.zeros((_NPAD - _NTOT,), jnp.int32)]
    )
    idx2 = idx_all.reshape(_EIDX_R, 128)
    flat_t, eidx2 = _detile(node_embedding_var, idx2)
    eidx = eidx2.reshape(-1)
    gathered = _sc_gather(flat_t, eidx)
    g2 = gathered.reshape(_D, _NPAD)
    nt = g2[:, :_B]
    wt = g2[:, _B:_B * (1 + _WALK)]
    gt = g2[:, _B * (1 + _WALK):_NTOT]

    net, loss = pl.pallas_call(
        _stats_body,
        out_shape=(
            jax.ShapeDtypeStruct((_D, _B), jnp.float32),
            jax.ShapeDtypeStruct((1, 1), jnp.float32),
        ),
        out_specs=(
            pl.BlockSpec(memory_space=pltpu.VMEM),
            pl.BlockSpec(memory_space=pltpu.SMEM),
        ),
    )(nt, wt, gt)

    ne = jnp.transpose(net)
    edge_t = pl.pallas_call(
        _edge_body,
        grid=(16,),
        in_specs=[
            pl.BlockSpec((64, _D), lambda i: (i, 0)),
            pl.BlockSpec((_D, _B), lambda i: (0, 0)),
        ],
        out_specs=pl.BlockSpec((64, _D, _B), lambda i: (i, 0, 0)),
        out_shape=jax.ShapeDtypeStruct((_B, _D, _B), jnp.float32),
    )(ne, net)
    edge = jnp.transpose(edge_t, (0, 2, 1))
    return loss[0, 0], ne, edge
